# baseline (speedup 1.0000x reference)
import functools

import numpy as np
import jax
import jax.numpy as jnp
from jax import lax
from jax.experimental import pallas as pl
from jax.experimental.pallas import tpu as pltpu

_BASE = 2


def _parity_taps(rs):
    return {
        0: [(_BASE + 0, 2, 2), (_BASE + 2, 2, 0),
            (_BASE + rs, 0, 2), (_BASE + rs + 2, 0, 0),
            (_BASE + 1, 2, 1), (_BASE + rs + 1, 0, 1)],
        1: [(_BASE + rs, 1, 2), (_BASE + rs + 2, 1, 0),
            (_BASE + rs + 1, 1, 1)],
    }


def _prep_body(x_ref, s2_ref, o_ref, *, h_in, rs):
    o_ref[...] = jnp.zeros_like(o_ref)
    xin = jnp.maximum(x_ref[0], 0.0).astype(jnp.bfloat16)
    for a in range(1, h_in + 1):
        row = xin[:, a - 1, :]
        dil = jnp.dot(row, s2_ref[...],
                      preferred_element_type=jnp.float32)
        o_ref[0, :row.shape[0], _BASE + a * rs:_BASE + (a + 1) * rs] = (
            dil.astype(o_ref.dtype))


def _parity_conv(d_ref, w0_ref, w1_ref, *, tile_m, rs):
    taps = _parity_taps(rs)
    zs = []
    for par, w_ref in ((0, w0_ref), (1, w1_ref)):
        xcat = jnp.concatenate(
            [d_ref[0, :, off:off + tile_m] for (off, _, _) in taps[par]],
            axis=0)
        zs.append(jnp.dot(w_ref[...], xcat, preferred_element_type=jnp.float32))
    return zs


def _stats_body(d_ref, w0, w1, o_ref, *, tile_m, rs):
    z0, z1 = _parity_conv(d_ref, w0, w1, tile_m=tile_m, rs=rs)
    o_ref[0, 0, :] = jnp.sum(z0, axis=1) + jnp.sum(z1, axis=1)
    o_ref[0, 1, :] = jnp.sum(z0 * z0, axis=1) + jnp.sum(z1 * z1, axis=1)


def _norm_body(d_ref, w0, w1, ss_ref, o_ref, *, tile_m, rs, z_len):
    z0, z1 = _parity_conv(d_ref, w0, w1, tile_m=tile_m, rs=rs)
    sc = ss_ref[:, 0:1]
    sh = ss_ref[:, 1:2]
    o_ref[0, 0, :, :] = (z0[:, :z_len] * sc + sh).astype(o_ref.dtype)
    o_ref[0, 1, :, :] = (z1[:, :z_len] * sc + sh).astype(o_ref.dtype)


@functools.partial(jax.jit, static_argnames=("eps",))
def _run(x, w, gamma, beta, *, eps=1e-5):
    N, Cin, H, W = x.shape
    Cin_w, Cout, K, K2 = w.shape
    assert Cin == Cin_w and K == 3 and K2 == 3
    s = 2
    Hg, Wg = H + 1, W + 1
    Ho, Wo = (H - 1) * s + K, (W - 1) * s + K
    Cin_p = 8 * (-(-Cin // 8))
    f32 = jnp.float32
    bf16 = jnp.bfloat16

    RS = s * Wg + 2
    Z = Hg * RS
    TMz = 128 * (-(-Z // 128))
    max_off = _BASE + RS + 2
    DL = 128 * (-(-(max_off + TMz) // 128))

    s2 = np.zeros((W, RS), np.float32)
    for b in range(W):
        s2[b, 2 * (b + 1)] = 1.0
    s2 = jnp.asarray(s2, bf16)

    wt = w.astype(f32)
    w_par = []
    for par in (0, 1):
        blocks = []
        for (_, kh, kw) in _parity_taps(RS)[par]:
            blk = jnp.transpose(wt[:, :, kh, kw], (1, 0))
            if Cin_p != Cin:
                blk = jnp.pad(blk, ((0, 0), (0, Cin_p - Cin)))
            blocks.append(blk)
        w_par.append(jnp.concatenate(blocks, axis=1).astype(bf16))

    cparams = pltpu.CompilerParams(dimension_semantics=("parallel",),
                                   vmem_limit_bytes=56 * 1024 * 1024)
    conv_flops = 2 * 9 * Cout * Cin_p * TMz * N

    d = pl.pallas_call(
        functools.partial(_prep_body, h_in=H, rs=RS),
        out_shape=jax.ShapeDtypeStruct((N, Cin_p, DL), bf16),
        grid=(N,),
        in_specs=[pl.BlockSpec((1, Cin, H, W), lambda n: (n, 0, 0, 0)),
                  pl.BlockSpec(s2.shape, lambda n: (0, 0))],
        out_specs=pl.BlockSpec((1, Cin_p, DL), lambda n: (n, 0, 0)),
        compiler_params=cparams,
        cost_estimate=pl.CostEstimate(
            flops=2 * N * Cin * H * W * RS, transcendentals=0,
            bytes_accessed=N * (Cin * H * W * 4 + Cin_p * DL * 2)),
    )(x, s2)

    dspec = pl.BlockSpec((1, Cin_p, DL), lambda n: (n, 0, 0))
    wspecs = [pl.BlockSpec(wp.shape, lambda n: (0, 0)) for wp in w_par]

    stats = pl.pallas_call(
        functools.partial(_stats_body, tile_m=TMz, rs=RS),
        out_shape=jax.ShapeDtypeStruct((N, 2, Cout), f32),
        grid=(N,),
        in_specs=[dspec] + wspecs,
        out_specs=pl.BlockSpec((1, 2, Cout), lambda n: (n, 0, 0)),
        compiler_params=cparams,
        cost_estimate=pl.CostEstimate(
            flops=conv_flops, transcendentals=0,
            bytes_accessed=N * Cin_p * DL * 2 + N * 2 * Cout * 4),
    )(d, *w_par)

    csum = jnp.sum(stats, axis=0)
    inv_count = 1.0 / float(N * Ho * Wo)
    mean = csum[0] * inv_count
    var = jnp.maximum(csum[1] * inv_count - mean * mean, 0.0)
    scale = gamma.astype(f32) * lax.rsqrt(var + float(eps))
    shift = beta.astype(f32) - mean * scale
    ss = jnp.stack([scale, shift], axis=1)

    y3 = pl.pallas_call(
        functools.partial(_norm_body, tile_m=TMz, rs=RS, z_len=Z),
        out_shape=jax.ShapeDtypeStruct((N, 2, Cout, Z), bf16),
        grid=(N,),
        in_specs=[dspec] + wspecs + [pl.BlockSpec((Cout, 2), lambda n: (0, 0))],
        out_specs=pl.BlockSpec((1, 2, Cout, Z), lambda n: (n, 0, 0, 0)),
        compiler_params=cparams,
        cost_estimate=pl.CostEstimate(
            flops=conv_flops, transcendentals=0,
            bytes_accessed=N * (Cin_p * DL * 2 + 2 * Cout * Z * 2)),
    )(d, *w_par, ss)

    yv = y3.reshape(N, 2, Cout, Hg, RS)
    y = jnp.transpose(yv, (0, 2, 3, 1, 4)).reshape(N, Cout, 2 * Hg, RS)
    return y[:, :, :Ho, :Wo].astype(f32)


def kernel(x, w, gamma, beta):
    return _run(x, w, gamma, beta)


# speedup vs baseline: 1.2754x; 1.2754x over previous
import functools

import numpy as np
import jax
import jax.numpy as jnp
from jax import lax
from jax.experimental import pallas as pl
from jax.experimental.pallas import tpu as pltpu

_SHIFTS = ((0, 0), (0, 1), (1, 0), (1, 1))
_PHASE_TAPS = {
    0: [(0, 2, 2), (1, 2, 0), (2, 0, 2), (3, 0, 0)],
    1: [(1, 2, 1), (3, 0, 1)],
    2: [(2, 1, 2), (3, 1, 0)],
    3: [(3, 1, 1)],
}


def _prep_body(x_ref, o_ref, *, hg, wg, w_in):
    o_ref[...] = jnp.zeros_like(o_ref)
    xin = jnp.maximum(x_ref[0], 0.0).astype(o_ref.dtype)
    for a in range(1, hg):
        o_ref[0, :xin.shape[0], a * wg + 1:a * wg + 1 + w_in] = xin[:, a - 1, :]


def _phase_conv(xf_ref, w_refs, *, offsets, tile_m):
    xs = [xf_ref[0, :, off:off + tile_m] for off in offsets]
    ys = []
    for p in range(4):
        sidx = [t[0] for t in _PHASE_TAPS[p]]
        xcat = xs[sidx[0]] if len(sidx) == 1 else jnp.concatenate(
            [xs[i] for i in sidx], axis=0)
        ys.append(jnp.dot(w_refs[p][...], xcat,
                          preferred_element_type=jnp.float32))
    return ys


def _stats_body(xf_ref, w0, w1, w2, w3, o_ref, *, offsets, tile_m, cout):
    ys = _phase_conv(xf_ref, (w0, w1, w2, w3), offsets=offsets, tile_m=tile_m)
    for p, y in enumerate(ys):
        o_ref[0, 0, p * cout:(p + 1) * cout] = jnp.sum(y, axis=1)
        o_ref[0, 1, p * cout:(p + 1) * cout] = jnp.sum(y * y, axis=1)


def _norm_body(xf_ref, w0, w1, w2, w3, ss_ref, o_ref, *, offsets, tile_m, cout):
    ys = _phase_conv(xf_ref, (w0, w1, w2, w3), offsets=offsets, tile_m=tile_m)
    for p, y in enumerate(ys):
        sc = ss_ref[p * cout:(p + 1) * cout, 0:1]
        sh = ss_ref[p * cout:(p + 1) * cout, 1:2]
        o_ref[0, p * cout:(p + 1) * cout, :] = (y * sc + sh).astype(o_ref.dtype)


@functools.partial(jax.jit, static_argnames=("eps",))
def _run(x, w, gamma, beta, *, eps=1e-5):
    N, Cin, H, W = x.shape
    Cin_w, Cout, K, K2 = w.shape
    assert Cin == Cin_w and K == 3 and K2 == 3
    s = 2
    Hg, Wg = H + 1, W + 1
    Ho, Wo = (H - 1) * s + K, (W - 1) * s + K
    Mn = Hg * Wg
    Cin_p = 8 * (-(-Cin // 8))
    f32 = jnp.float32
    bf16 = jnp.bfloat16

    TM = 128 * (-(-Mn // 128))
    Mn_pad = TM + 128
    assert Wg + 1 <= 128
    offsets = tuple(th * Wg + tw for th, tw in _SHIFTS)
    PCout = 4 * Cout

    wt = w.astype(f32)
    w_packed = []
    for p in range(4):
        blocks = []
        for (_, kh, kw) in _PHASE_TAPS[p]:
            blk = jnp.transpose(wt[:, :, kh, kw], (1, 0))
            if Cin_p != Cin:
                blk = jnp.pad(blk, ((0, 0), (0, Cin_p - Cin)))
            blocks.append(blk)
        w_packed.append(jnp.concatenate(blocks, axis=1).astype(bf16))

    cparams = pltpu.CompilerParams(dimension_semantics=("parallel",),
                                   vmem_limit_bytes=56 * 1024 * 1024)
    conv_flops = 2 * 9 * Cout * Cin_p * TM * N

    xf = pl.pallas_call(
        functools.partial(_prep_body, hg=Hg, wg=Wg, w_in=W),
        out_shape=jax.ShapeDtypeStruct((N, Cin_p, Mn_pad), bf16),
        grid=(N,),
        in_specs=[pl.BlockSpec((1, Cin, H, W), lambda n: (n, 0, 0, 0))],
        out_specs=pl.BlockSpec((1, Cin_p, Mn_pad), lambda n: (n, 0, 0)),
        compiler_params=cparams,
        cost_estimate=pl.CostEstimate(
            flops=N * Cin * H * W, transcendentals=0,
            bytes_accessed=N * (Cin * H * W * 4 + Cin_p * Mn_pad * 2)),
    )(x)

    xfspec = pl.BlockSpec((1, Cin_p, Mn_pad), lambda n: (n, 0, 0))
    wspecs = [pl.BlockSpec(wp.shape, lambda n: (0, 0)) for wp in w_packed]

    stats = pl.pallas_call(
        functools.partial(_stats_body, offsets=offsets, tile_m=TM, cout=Cout),
        out_shape=jax.ShapeDtypeStruct((N, 2, PCout), f32),
        grid=(N,),
        in_specs=[xfspec] + wspecs,
        out_specs=pl.BlockSpec((1, 2, PCout), lambda n: (n, 0, 0)),
        compiler_params=cparams,
        cost_estimate=pl.CostEstimate(
            flops=conv_flops, transcendentals=0,
            bytes_accessed=N * Cin_p * Mn_pad * 2 + N * 2 * PCout * 4),
    )(xf, *w_packed)

    sums = jnp.sum(stats, axis=0)
    csum = sums.reshape(2, 4, Cout).sum(axis=1)
    inv_count = 1.0 / float(N * Ho * Wo)
    mean = csum[0] * inv_count
    var = jnp.maximum(csum[1] * inv_count - mean * mean, 0.0)
    scale = gamma.astype(f32) * lax.rsqrt(var + float(eps))
    shift = beta.astype(f32) - mean * scale
    ss = jnp.tile(jnp.stack([scale, shift], axis=1), (4, 1))

    y2 = pl.pallas_call(
        functools.partial(_norm_body, offsets=offsets, tile_m=TM, cout=Cout),
        out_shape=jax.ShapeDtypeStruct((N, PCout, TM), bf16),
        grid=(N,),
        in_specs=[xfspec] + wspecs + [pl.BlockSpec((PCout, 2),
                                                   lambda n: (0, 0))],
        out_specs=pl.BlockSpec((1, PCout, TM), lambda n: (n, 0, 0)),
        compiler_params=cparams,
        cost_estimate=pl.CostEstimate(
            flops=conv_flops, transcendentals=0,
            bytes_accessed=N * (Cin_p * Mn_pad * 2 + PCout * TM * 2)),
    )(xf, *w_packed, ss)

    yv = y2[:, :, :Mn].reshape(N, s, s, Cout, Hg, Wg)
    y = jnp.transpose(yv, (0, 3, 4, 1, 5, 2)).reshape(N, Cout, Hg * s, Wg * s)
    return y[:, :, :Ho, :Wo].astype(f32)


def kernel(x, w, gamma, beta):
    return _run(x, w, gamma, beta)


# speedup vs baseline: 1.2789x; 1.0028x over previous
import functools

import numpy as np
import jax
import jax.numpy as jnp
from jax import lax
from jax.experimental import pallas as pl
from jax.experimental.pallas import tpu as pltpu

_SHIFTS = ((0, 0), (0, 1), (1, 0), (1, 1))
_PHASE_TAPS = {
    0: [(0, 2, 2), (1, 2, 0), (2, 0, 2), (3, 0, 0)],
    1: [(1, 2, 1), (3, 0, 1)],
    2: [(2, 1, 2), (3, 1, 0)],
    3: [(3, 1, 1)],
}


def _prep_body(x_ref, o_ref, *, hg, wg, w_in):
    o_ref[...] = jnp.zeros_like(o_ref)
    xin = jnp.maximum(x_ref[0], 0.0).astype(o_ref.dtype)
    for a in range(1, hg):
        o_ref[0, :xin.shape[0], a * wg + 1:a * wg + 1 + w_in] = xin[:, a - 1, :]


def _phase_conv(xf_ref, w_refs, *, offsets, tile_m):
    xs = [xf_ref[0, :, off:off + tile_m] for off in offsets]
    ys = []
    for p in range(4):
        sidx = [t[0] for t in _PHASE_TAPS[p]]
        xcat = xs[sidx[0]] if len(sidx) == 1 else jnp.concatenate(
            [xs[i] for i in sidx], axis=0)
        ys.append(jnp.dot(w_refs[p][...], xcat,
                          preferred_element_type=jnp.float32))
    return ys


def _prep_stats_body(x_ref, w0, w1, w2, w3, o_ref, xf_ref, *,
                     offsets, tile_m, cout, hg, wg, w_in):
    _prep_body(x_ref, xf_ref, hg=hg, wg=wg, w_in=w_in)
    ys = _phase_conv(xf_ref, (w0, w1, w2, w3), offsets=offsets, tile_m=tile_m)
    for p, y in enumerate(ys):
        o_ref[0, 0, p * cout:(p + 1) * cout] = jnp.sum(y, axis=1)
        o_ref[0, 1, p * cout:(p + 1) * cout] = jnp.sum(y * y, axis=1)


def _norm_body(xf_ref, w0, w1, w2, w3, ss_ref, o_ref, *, offsets, tile_m, cout):
    ys = _phase_conv(xf_ref, (w0, w1, w2, w3), offsets=offsets, tile_m=tile_m)
    for p, y in enumerate(ys):
        sc = ss_ref[p * cout:(p + 1) * cout, 0:1]
        sh = ss_ref[p * cout:(p + 1) * cout, 1:2]
        o_ref[0, p * cout:(p + 1) * cout, :] = (y * sc + sh).astype(o_ref.dtype)


@functools.partial(jax.jit, static_argnames=("eps",))
def _run(x, w, gamma, beta, *, eps=1e-5):
    N, Cin, H, W = x.shape
    Cin_w, Cout, K, K2 = w.shape
    assert Cin == Cin_w and K == 3 and K2 == 3
    s = 2
    Hg, Wg = H + 1, W + 1
    Ho, Wo = (H - 1) * s + K, (W - 1) * s + K
    Mn = Hg * Wg
    Cin_p = 8 * (-(-Cin // 8))
    f32 = jnp.float32
    bf16 = jnp.bfloat16

    TM = 128 * (-(-Mn // 128))
    Mn_pad = TM + 128
    assert Wg + 1 <= 128
    offsets = tuple(th * Wg + tw for th, tw in _SHIFTS)
    PCout = 4 * Cout

    wt = w.astype(f32)
    w_packed = []
    for p in range(4):
        blocks = []
        for (_, kh, kw) in _PHASE_TAPS[p]:
            blk = jnp.transpose(wt[:, :, kh, kw], (1, 0))
            if Cin_p != Cin:
                blk = jnp.pad(blk, ((0, 0), (0, Cin_p - Cin)))
            blocks.append(blk)
        w_packed.append(jnp.concatenate(blocks, axis=1).astype(bf16))

    cparams = pltpu.CompilerParams(dimension_semantics=("parallel",),
                                   vmem_limit_bytes=56 * 1024 * 1024)
    conv_flops = 2 * 9 * Cout * Cin_p * TM * N

    xfspec = pl.BlockSpec((1, Cin_p, Mn_pad), lambda n: (n, 0, 0))
    wspecs = [pl.BlockSpec(wp.shape, lambda n: (0, 0)) for wp in w_packed]

    stats, xf = pl.pallas_call(
        functools.partial(_prep_stats_body, offsets=offsets, tile_m=TM,
                          cout=Cout, hg=Hg, wg=Wg, w_in=W),
        out_shape=[jax.ShapeDtypeStruct((N, 2, PCout), f32),
                   jax.ShapeDtypeStruct((N, Cin_p, Mn_pad), bf16)],
        grid=(N,),
        in_specs=[pl.BlockSpec((1, Cin, H, W), lambda n: (n, 0, 0, 0))] + wspecs,
        out_specs=[pl.BlockSpec((1, 2, PCout), lambda n: (n, 0, 0)), xfspec],
        compiler_params=cparams,
        cost_estimate=pl.CostEstimate(
            flops=conv_flops, transcendentals=0,
            bytes_accessed=N * (Cin * H * W * 4 + Cin_p * Mn_pad * 2
                                + 2 * PCout * 4)),
    )(x, *w_packed)

    sums = jnp.sum(stats, axis=0)
    csum = sums.reshape(2, 4, Cout).sum(axis=1)
    inv_count = 1.0 / float(N * Ho * Wo)
    mean = csum[0] * inv_count
    var = jnp.maximum(csum[1] * inv_count - mean * mean, 0.0)
    scale = gamma.astype(f32) * lax.rsqrt(var + float(eps))
    shift = beta.astype(f32) - mean * scale
    ss = jnp.tile(jnp.stack([scale, shift], axis=1), (4, 1))

    y2 = pl.pallas_call(
        functools.partial(_norm_body, offsets=offsets, tile_m=TM, cout=Cout),
        out_shape=jax.ShapeDtypeStruct((N, PCout, TM), bf16),
        grid=(N,),
        in_specs=[xfspec] + wspecs + [pl.BlockSpec((PCout, 2),
                                                   lambda n: (0, 0))],
        out_specs=pl.BlockSpec((1, PCout, TM), lambda n: (n, 0, 0)),
        compiler_params=cparams,
        cost_estimate=pl.CostEstimate(
            flops=conv_flops, transcendentals=0,
            bytes_accessed=N * (Cin_p * Mn_pad * 2 + PCout * TM * 2)),
    )(xf, *w_packed, ss)

    yv = y2[:, :, :Mn].reshape(N, s, s, Cout, Hg, Wg)
    y = jnp.transpose(yv, (0, 3, 4, 1, 5, 2)).reshape(N, Cout, Hg * s, Wg * s)
    return y[:, :, :Ho, :Wo].astype(f32)


def kernel(x, w, gamma, beta):
    return _run(x, w, gamma, beta)
